# initial kernel scaffold (unmeasured)
import jax
import jax.numpy as jnp
from jax import lax
from jax.experimental import pallas as pl
from jax.experimental.pallas import tpu as pltpu


def kernel(x, W):
    t, d = x.shape
    _, v = W.shape
    V = 2 * v

    def body(x_ref, w_ref, out_ref, s_send, s_recv,
             stat_send_sem, stat_recv_sem, big_send_sem, big_recv_sem):
        my_x = lax.axis_index("x")
        my_y = lax.axis_index("y")
        nbr = (1 - my_x, my_y)

        e = jnp.exp(jnp.dot(x_ref[...], w_ref[...],
                            preferred_element_type=jnp.float32))
        s = jnp.sum(e, axis=1, keepdims=True)
        out_ref[:, pl.ds(my_x * v, v)] = e
        s_send[...] = jnp.broadcast_to(s, s_send.shape)

        stat_rdma = pltpu.make_async_remote_copy(
            src_ref=s_send, dst_ref=s_recv,
            send_sem=stat_send_sem, recv_sem=stat_recv_sem,
            device_id=nbr, device_id_type=pl.DeviceIdType.MESH)
        stat_rdma.start()
        stat_rdma.wait()

        inv = 1.0 / (s + s_recv[:, 0:1])
        out_ref[:, pl.ds(my_x * v, v)] = out_ref[:, pl.ds(my_x * v, v)] * inv

        big_rdma = pltpu.make_async_remote_copy(
            src_ref=out_ref.at[:, pl.ds(my_x * v, v)],
            dst_ref=out_ref.at[:, pl.ds(my_x * v, v)],
            send_sem=big_send_sem, recv_sem=big_recv_sem,
            device_id=nbr, device_id_type=pl.DeviceIdType.MESH)
        big_rdma.start()
        big_rdma.wait()

    return pl.pallas_call(
        body,
        out_shape=jax.ShapeDtypeStruct((t, V), jnp.float32),
        in_specs=[pl.BlockSpec(memory_space=pltpu.VMEM),
                  pl.BlockSpec(memory_space=pltpu.VMEM)],
        out_specs=pl.BlockSpec(memory_space=pltpu.VMEM),
        scratch_shapes=[
            pltpu.VMEM((t, 128), jnp.float32),
            pltpu.VMEM((t, 128), jnp.float32),
            pltpu.SemaphoreType.DMA,
            pltpu.SemaphoreType.DMA,
            pltpu.SemaphoreType.DMA,
            pltpu.SemaphoreType.DMA,
        ],
    )(x, W)


# baseline (device time: 243580 ns/iter reference)
import jax
import jax.numpy as jnp
from jax import lax
from jax.experimental import pallas as pl
from jax.experimental.pallas import tpu as pltpu

K = 8


def kernel(x, W):
    t, d = x.shape
    _, v = W.shape
    V = 2 * v
    ck = v // K

    def body(x_ref, w_hbm, out_ref, wbuf, s_send, s_recv,
             wsems, stat_send_sem, stat_recv_sem, big_send_sem, big_recv_sem):
        my_x = lax.axis_index("x")
        my_y = lax.axis_index("y")
        nbr = (1 - my_x, my_y)

        def start_wcopy(k, slot):
            c = pltpu.make_async_copy(
                w_hbm.at[:, pl.ds(k * ck, ck)], wbuf.at[slot], wsems.at[slot])
            c.start()
            return c

        pending = start_wcopy(0, 0)
        s = jnp.zeros((t, 1), jnp.float32)
        for k in range(K):
            slot = k % 2
            nxt = start_wcopy(k + 1, 1 - slot) if k + 1 < K else None
            pending.wait()
            e = jnp.exp(jnp.dot(x_ref[...], wbuf[slot],
                                preferred_element_type=jnp.float32))
            out_ref[:, pl.ds(my_x * v + k * ck, ck)] = e
            s = s + jnp.sum(e, axis=1, keepdims=True)
            pending = nxt

        s_send[...] = jnp.broadcast_to(s, s_send.shape)
        stat_rdma = pltpu.make_async_remote_copy(
            src_ref=s_send, dst_ref=s_recv,
            send_sem=stat_send_sem, recv_sem=stat_recv_sem,
            device_id=nbr, device_id_type=pl.DeviceIdType.MESH)
        stat_rdma.start()
        stat_rdma.wait()

        inv = 1.0 / (s + s_recv[:, 0:1])
        out_ref[:, pl.ds(my_x * v, v)] = out_ref[:, pl.ds(my_x * v, v)] * inv

        big_rdma = pltpu.make_async_remote_copy(
            src_ref=out_ref.at[:, pl.ds(my_x * v, v)],
            dst_ref=out_ref.at[:, pl.ds(my_x * v, v)],
            send_sem=big_send_sem, recv_sem=big_recv_sem,
            device_id=nbr, device_id_type=pl.DeviceIdType.MESH)
        big_rdma.start()
        big_rdma.wait()

    return pl.pallas_call(
        body,
        out_shape=jax.ShapeDtypeStruct((t, V), jnp.float32),
        in_specs=[pl.BlockSpec(memory_space=pltpu.VMEM),
                  pl.BlockSpec(memory_space=pl.ANY)],
        out_specs=pl.BlockSpec(memory_space=pltpu.VMEM),
        scratch_shapes=[
            pltpu.VMEM((2, d, ck), jnp.float32),
            pltpu.VMEM((t, 128), jnp.float32),
            pltpu.VMEM((t, 128), jnp.float32),
            pltpu.SemaphoreType.DMA((2,)),
            pltpu.SemaphoreType.DMA,
            pltpu.SemaphoreType.DMA,
            pltpu.SemaphoreType.DMA,
            pltpu.SemaphoreType.DMA,
        ],
        compiler_params=pltpu.CompilerParams(
            vmem_limit_bytes=56 * 1024 * 1024),
    )(x, W)


# device time: 231303 ns/iter; 1.0531x vs baseline; 1.0531x over previous
import jax
import jax.numpy as jnp
from jax import lax
from jax.experimental import pallas as pl
from jax.experimental.pallas import tpu as pltpu

K = 8


def kernel(x, W):
    t, d = x.shape
    _, v = W.shape
    V = 2 * v
    ck = v // K

    def body(x_ref, w_hbm, out_ref, wbuf, s_send, s_recv,
             wsems, stat_send_sem, stat_recv_sem, big_send_sems, big_recv_sems):
        my_x = lax.axis_index("x")
        my_y = lax.axis_index("y")
        nbr = (1 - my_x, my_y)

        def start_wcopy(k, slot):
            c = pltpu.make_async_copy(
                w_hbm.at[:, pl.ds(k * ck, ck)], wbuf.at[slot], wsems.at[slot])
            c.start()
            return c

        def chunk_rdma(k):
            sl = pl.ds(my_x * v + k * ck, ck)
            return pltpu.make_async_remote_copy(
                src_ref=out_ref.at[:, sl], dst_ref=out_ref.at[:, sl],
                send_sem=big_send_sems.at[k], recv_sem=big_recv_sems.at[k],
                device_id=nbr, device_id_type=pl.DeviceIdType.MESH)

        pending = start_wcopy(0, 0)
        s = jnp.zeros((t, 1), jnp.float32)
        rdmas = []
        for k in range(K):
            slot = k % 2
            nxt = start_wcopy(k + 1, 1 - slot) if k + 1 < K else None
            pending.wait()
            e = jnp.exp(jnp.dot(x_ref[...], wbuf[slot],
                                preferred_element_type=jnp.float32))
            out_ref[:, pl.ds(my_x * v + k * ck, ck)] = e
            s = s + jnp.sum(e, axis=1, keepdims=True)
            r = chunk_rdma(k)
            r.start()
            rdmas.append(r)
            pending = nxt

        s_send[...] = jnp.broadcast_to(s, s_send.shape)
        stat_rdma = pltpu.make_async_remote_copy(
            src_ref=s_send, dst_ref=s_recv,
            send_sem=stat_send_sem, recv_sem=stat_recv_sem,
            device_id=nbr, device_id_type=pl.DeviceIdType.MESH)
        stat_rdma.start()
        stat_rdma.wait()
        inv = 1.0 / (s + s_recv[:, 0:1])

        for k in range(K):
            mine = pl.ds(my_x * v + k * ck, ck)
            theirs = pl.ds((1 - my_x) * v + k * ck, ck)
            rdmas[k].wait_send()
            out_ref[:, mine] = out_ref[:, mine] * inv
            rdmas[k].wait_recv()
            out_ref[:, theirs] = out_ref[:, theirs] * inv

    return pl.pallas_call(
        body,
        out_shape=jax.ShapeDtypeStruct((t, V), jnp.float32),
        in_specs=[pl.BlockSpec(memory_space=pltpu.VMEM),
                  pl.BlockSpec(memory_space=pl.ANY)],
        out_specs=pl.BlockSpec(memory_space=pltpu.VMEM),
        scratch_shapes=[
            pltpu.VMEM((2, d, ck), jnp.float32),
            pltpu.VMEM((t, 128), jnp.float32),
            pltpu.VMEM((t, 128), jnp.float32),
            pltpu.SemaphoreType.DMA((2,)),
            pltpu.SemaphoreType.DMA,
            pltpu.SemaphoreType.DMA,
            pltpu.SemaphoreType.DMA((K,)),
            pltpu.SemaphoreType.DMA((K,)),
        ],
        compiler_params=pltpu.CompilerParams(
            vmem_limit_bytes=56 * 1024 * 1024),
    )(x, W)


# device time: 138894 ns/iter; 1.7537x vs baseline; 1.6653x over previous
import jax
import jax.numpy as jnp
from jax import lax
from jax.experimental import pallas as pl
from jax.experimental.pallas import tpu as pltpu

K = 8
DEFER = 2


def kernel(x, W):
    t, d = x.shape
    _, v = W.shape
    V = 2 * v
    ck = v // K

    def body(x_ref, w_hbm, out_ref, wbuf, send_buf, recv_buf, s_send, s_recv,
             wsems, stat_send_sem, stat_recv_sem, big_send_sems, big_recv_sems):
        my_x = lax.axis_index("x")
        my_y = lax.axis_index("y")
        nbr = (1 - my_x, my_y)

        def start_wcopy(k, slot):
            c = pltpu.make_async_copy(
                w_hbm.at[:, pl.ds(k * ck, ck)], wbuf.at[slot], wsems.at[slot])
            c.start()
            return c

        def chunk_rdma(k):
            sl = pl.ds(k * ck, ck)
            return pltpu.make_async_remote_copy(
                src_ref=send_buf.at[:, sl], dst_ref=recv_buf.at[:, sl],
                send_sem=big_send_sems.at[k], recv_sem=big_recv_sems.at[k],
                device_id=nbr, device_id_type=pl.DeviceIdType.MESH)

        pending = start_wcopy(0, 0)
        s = jnp.zeros((t, 1), jnp.float32)
        rdmas = []
        for k in range(K):
            slot = k % 2
            nxt = start_wcopy(k + 1, 1 - slot) if k + 1 < K else None
            pending.wait()
            e = jnp.exp(jnp.dot(x_ref[...], wbuf[slot],
                                preferred_element_type=jnp.float32))
            out_ref[:, pl.ds(my_x * v + k * ck, ck)] = e
            send_buf[:, pl.ds(k * ck, ck)] = e.astype(jnp.bfloat16)
            s = s + jnp.sum(e, axis=1, keepdims=True)
            r = chunk_rdma(k)
            rdmas.append(r)
            if k < K - DEFER:
                r.start()
            pending = nxt

        s_send[...] = jnp.broadcast_to(s, s_send.shape)
        stat_rdma = pltpu.make_async_remote_copy(
            src_ref=s_send, dst_ref=s_recv,
            send_sem=stat_send_sem, recv_sem=stat_recv_sem,
            device_id=nbr, device_id_type=pl.DeviceIdType.MESH)
        stat_rdma.start()
        for k in range(K - DEFER, K):
            rdmas[k].start()

        stat_rdma.wait()
        inv = 1.0 / (s + s_recv[:, 0:1])

        for k in range(K):
            mine = pl.ds(my_x * v + k * ck, ck)
            out_ref[:, mine] = out_ref[:, mine] * inv

        for k in range(K):
            rdmas[k].wait_recv()
            sl = pl.ds(k * ck, ck)
            out_ref[:, pl.ds((1 - my_x) * v + k * ck, ck)] = (
                recv_buf[:, sl].astype(jnp.float32) * inv)

        for k in range(K):
            rdmas[k].wait_send()

    return pl.pallas_call(
        body,
        out_shape=jax.ShapeDtypeStruct((t, V), jnp.float32),
        in_specs=[pl.BlockSpec(memory_space=pltpu.VMEM),
                  pl.BlockSpec(memory_space=pl.ANY)],
        out_specs=pl.BlockSpec(memory_space=pltpu.VMEM),
        scratch_shapes=[
            pltpu.VMEM((2, d, ck), jnp.float32),
            pltpu.VMEM((t, v), jnp.bfloat16),
            pltpu.VMEM((t, v), jnp.bfloat16),
            pltpu.VMEM((t, 128), jnp.float32),
            pltpu.VMEM((t, 128), jnp.float32),
            pltpu.SemaphoreType.DMA((2,)),
            pltpu.SemaphoreType.DMA,
            pltpu.SemaphoreType.DMA,
            pltpu.SemaphoreType.DMA((K,)),
            pltpu.SemaphoreType.DMA((K,)),
        ],
        compiler_params=pltpu.CompilerParams(
            vmem_limit_bytes=60 * 1024 * 1024),
    )(x, W)


# device time: 138770 ns/iter; 1.7553x vs baseline; 1.0009x over previous
import jax
import jax.numpy as jnp
from jax import lax
from jax.experimental import pallas as pl
from jax.experimental.pallas import tpu as pltpu

K = 8
DEFER = 2


def kernel(x, W):
    t, d = x.shape
    _, v = W.shape
    V = 2 * v
    ck = v // K

    def body(x_ref, w_hbm, out_ref, wbuf, send_buf, recv_buf, s_send, s_recv,
             wsems, stat_send_sem, stat_recv_sem, big_send_sems, big_recv_sems):
        my_x = lax.axis_index("x")
        my_y = lax.axis_index("y")
        nbr = (1 - my_x, my_y)

        def start_wcopy(k, slot):
            c = pltpu.make_async_copy(
                w_hbm.at[:, pl.ds(k * ck, ck)], wbuf.at[slot], wsems.at[slot])
            c.start()
            return c

        def chunk_rdma(k):
            sl = pl.ds(k * ck, ck)
            return pltpu.make_async_remote_copy(
                src_ref=send_buf.at[:, sl], dst_ref=recv_buf.at[:, sl],
                send_sem=big_send_sems.at[k], recv_sem=big_recv_sems.at[k],
                device_id=nbr, device_id_type=pl.DeviceIdType.MESH)

        pending = start_wcopy(0, 0)
        s = jnp.zeros((t, 1), jnp.float32)
        rdmas = []
        for k in range(K):
            slot = k % 2
            nxt = start_wcopy(k + 1, 1 - slot) if k + 1 < K else None
            pending.wait()
            e = jnp.exp(jnp.dot(x_ref[...], wbuf[slot],
                                preferred_element_type=jnp.float32))
            send_buf[:, pl.ds(k * ck, ck)] = e.astype(jnp.bfloat16)
            s = s + jnp.sum(e, axis=1, keepdims=True)
            r = chunk_rdma(k)
            rdmas.append(r)
            if k < K - DEFER:
                r.start()
            pending = nxt

        s_send[...] = jnp.broadcast_to(s, s_send.shape)
        stat_rdma = pltpu.make_async_remote_copy(
            src_ref=s_send, dst_ref=s_recv,
            send_sem=stat_send_sem, recv_sem=stat_recv_sem,
            device_id=nbr, device_id_type=pl.DeviceIdType.MESH)
        stat_rdma.start()
        for k in range(K - DEFER, K):
            rdmas[k].start()

        stat_rdma.wait()
        inv = 1.0 / (s + s_recv[:, 0:1])

        for k in range(K):
            sl = pl.ds(k * ck, ck)
            out_ref[:, pl.ds(my_x * v + k * ck, ck)] = (
                send_buf[:, sl].astype(jnp.float32) * inv)

        for k in range(K):
            rdmas[k].wait_recv()
            sl = pl.ds(k * ck, ck)
            out_ref[:, pl.ds((1 - my_x) * v + k * ck, ck)] = (
                recv_buf[:, sl].astype(jnp.float32) * inv)

        for k in range(K):
            rdmas[k].wait_send()

    return pl.pallas_call(
        body,
        out_shape=jax.ShapeDtypeStruct((t, V), jnp.float32),
        in_specs=[pl.BlockSpec(memory_space=pltpu.VMEM),
                  pl.BlockSpec(memory_space=pl.ANY)],
        out_specs=pl.BlockSpec(memory_space=pltpu.VMEM),
        scratch_shapes=[
            pltpu.VMEM((2, d, ck), jnp.float32),
            pltpu.VMEM((t, v), jnp.bfloat16),
            pltpu.VMEM((t, v), jnp.bfloat16),
            pltpu.VMEM((t, 128), jnp.float32),
            pltpu.VMEM((t, 128), jnp.float32),
            pltpu.SemaphoreType.DMA((2,)),
            pltpu.SemaphoreType.DMA,
            pltpu.SemaphoreType.DMA,
            pltpu.SemaphoreType.DMA((K,)),
            pltpu.SemaphoreType.DMA((K,)),
        ],
        compiler_params=pltpu.CompilerParams(
            vmem_limit_bytes=60 * 1024 * 1024),
    )(x, W)
